# baseline (speedup 1.0000x reference)
import jax
import jax.numpy as jnp
import numpy as np
from jax.experimental import pallas as pl
from jax.experimental.pallas import tpu as pltpu

WIN = 16
DIM = 192
HEADS = 4
B, H, W = 2, 384, 384
HD = DIM // HEADS
P = WIN * WIN
KK = int((H // WIN) * (W // WIN) * 0.25)
N = B * KK
BAND = 16
SCALE = HD ** -0.5


def _rel_pos_index(w):
    coords = np.stack(np.meshgrid(np.arange(w), np.arange(w), indexing='ij'), axis=0)
    cq = coords.reshape(2, -1)
    ck = coords.reshape(2, -1)
    rel = cq[:, :, None] - ck[:, None, :]
    rel = rel.transpose(1, 2, 0) + (w - 1)
    return (rel[:, :, 0] * (2 * w - 1) + rel[:, :, 1]).astype(np.int32)


def _conv2d(x, w, b, stride=1, pad=1, groups=1):
    out = jax.lax.conv_general_dilated(
        x, w, window_strides=(stride, stride),
        padding=[(pad, pad), (pad, pad)],
        dimension_numbers=('NCHW', 'OIHW', 'NCHW'), feature_group_count=groups)
    return out + b[None, :, None, None]


def _silu(x):
    return x * jax.nn.sigmoid(x)


def _gather_copy(idx_ref, x_hbm, rext2, sems, i, slot):
    b = idx_ref[0, i]
    sh = idx_ref[3, i]
    sw = idx_ref[4, i]
    swa = jnp.minimum((sw // 8) * 8, W - 24)
    return pltpu.make_async_copy(
        x_hbm.at[b, pl.ds(sh, 17), pl.ds(swa, 24), :],
        rext2.at[slot], sems.at[slot])


def _attn_body(idx_ref, x_hbm, wqkv_ref, bqkv_ref, bias_ref,
               wout_ref, bout_ref, gd_ref, rext2, sems):
    i = pl.program_id(0)
    sw = idx_ref[4, i]
    dh = idx_ref[5, i]
    dw = idx_ref[6, i]
    hard = idx_ref[7, i]
    swa = jnp.minimum((sw // 8) * 8, W - 24)
    doff = sw - swa
    slot = jax.lax.rem(i, 3)

    @pl.when((i == 0) & (hard == 1))
    def _():
        _gather_copy(idx_ref, x_hbm, rext2, sems, i, slot).start()

    @pl.when((i == 0) & (1 < N) & (idx_ref[7, 1] == 1))
    def _():
        _gather_copy(idx_ref, x_hbm, rext2, sems, 1, 1).start()

    @pl.when((i + 2 < N) & (idx_ref[7, i + 2] == 1))
    def _():
        nslot = jax.lax.rem(i + 2, 3)
        _gather_copy(idx_ref, x_hbm, rext2, sems, i + 2, nslot).start()

    @pl.when(hard == 1)
    def _():
        _gather_copy(idx_ref, x_hbm, rext2, sems, i, slot).wait()
        R = pltpu.roll(rext2[slot], (24 - doff) % 24, 1)[:, 0:17, :]
        Rr = jnp.concatenate([R[1:17], R[16:17]], axis=0)
        Rc = jnp.concatenate([R[:, 1:17], R[:, 16:17]], axis=1)
        Rrc = jnp.concatenate([Rc[1:17], Rc[16:17]], axis=0)
        S = R + Rc + Rr + Rrc
        p00 = S[0:16, 0:16, :]
        p01 = S[0:16, 1:17, :]
        p10 = S[1:17, 0:16, :]
        p11 = S[1:17, 1:17, :]
        patch3 = 0.25 * jnp.where(dh == 0,
                                  jnp.where(dw == 0, p00, p01),
                                  jnp.where(dw == 0, p10, p11))
        patch = patch3.reshape(P, DIM)
        qkv = jnp.dot(patch, wqkv_ref[...],
                      preferred_element_type=jnp.float32) + bqkv_ref[0, :]
        q = qkv[:, 0:DIM]
        k = qkv[:, DIM:2 * DIM]
        v = qkv[:, 2 * DIM:3 * DIM]
        outs = []
        for hh in range(HEADS):
            sl = slice(hh * HD, (hh + 1) * HD)
            sc = jax.lax.dot_general(
                q[:, sl], k[:, sl], (((1,), (1,)), ((), ())),
                preferred_element_type=jnp.float32) * SCALE + bias_ref[hh, :, :]
            m = jnp.max(sc, axis=1, keepdims=True)
            e = jnp.exp(sc - m)
            s = jnp.sum(e, axis=1, keepdims=True)
            outs.append(jnp.dot(e, v[:, sl],
                                preferred_element_type=jnp.float32) / s)
        o = jnp.concatenate(outs, axis=1)
        after = jnp.dot(o, wout_ref[...],
                        preferred_element_type=jnp.float32) + bout_ref[0, :]
        gd_ref[0] = (after - patch).reshape(WIN, WIN, DIM)


def _score_body(x_hbm, w1_ref, b1_ref, out_ref, xs, sem0):
    bI = pl.program_id(0)
    j = pl.program_id(1)
    nj = pl.num_programs(1)
    r0 = j * BAND

    c_x = pltpu.make_async_copy(
        x_hbm.at[bI, pl.ds(r0, BAND), :, :], xs.at[pl.ds(1, BAND)], sem0)
    c_x.start()
    c_x.wait()

    @pl.when(j > 0)
    def _():
        t_x = pltpu.make_async_copy(
            x_hbm.at[bI, pl.ds(r0 - 1, 1), :, :], xs.at[pl.ds(0, 1)], sem0)
        t_x.start()
        t_x.wait()

    @pl.when(j == 0)
    def _():
        xs[0, :, :] = jnp.zeros((W, DIM), jnp.float32)

    @pl.when(j < nj - 1)
    def _():
        b_x = pltpu.make_async_copy(
            x_hbm.at[bI, pl.ds(r0 + BAND, 1), :, :],
            xs.at[pl.ds(BAND + 1, 1)], sem0)
        b_x.start()
        b_x.wait()

    @pl.when(j == nj - 1)
    def _():
        xs[BAND + 1, :, :] = jnp.zeros((W, DIM), jnp.float32)

    t = xs[...]
    z = jnp.zeros((BAND + 2, 1, DIM), jnp.float32)
    tl = jnp.concatenate([z, t[:, 0:W - 1, :]], axis=1)
    tr = jnp.concatenate([t[:, 1:W, :], z], axis=1)
    acc = jnp.zeros((BAND * W, 4), jnp.float32)
    for dr in range(3):
        rows = slice(dr, dr + BAND)
        for dc, src in ((0, tl), (1, t), (2, tr)):
            acc = acc + jnp.dot(src[rows].reshape(BAND * W, DIM),
                                w1_ref[(3 * dr + dc) * DIM:
                                       (3 * dr + dc + 1) * DIM, :],
                                preferred_element_type=jnp.float32)
    acc = acc + b1_ref[0]
    out_ref[0] = (acc * jax.nn.sigmoid(acc)).reshape(BAND, W, 4)


def _conv1_silu(x, w1r, b1, interpret=False):
    grid_spec = pltpu.PrefetchScalarGridSpec(
        num_scalar_prefetch=0,
        grid=(B, H // BAND),
        in_specs=[
            pl.BlockSpec(memory_space=pl.ANY),
            pl.BlockSpec((9 * DIM, 4), lambda b, j: (0, 0)),
            pl.BlockSpec((1, 4), lambda b, j: (0, 0)),
        ],
        out_specs=pl.BlockSpec((1, BAND, W, 4), lambda b, j: (b, j, 0, 0)),
        scratch_shapes=[
            pltpu.VMEM((BAND + 2, W, DIM), jnp.float32),
            pltpu.SemaphoreType.DMA,
        ],
    )
    return pl.pallas_call(
        _score_body,
        grid_spec=grid_spec,
        out_shape=jax.ShapeDtypeStruct((B, H, W, 4), jnp.float32),
        compiler_params=pltpu.CompilerParams(
            dimension_semantics=("parallel", "arbitrary")),
        interpret=interpret,
    )(x, w1r, b1)


ACCR = BAND + 32


def _fin_body(wdesc_ref, lo_ref, hi_ref, x_hbm, gd_hbm, wd_ref, bd_ref,
              out_ref, xs, acc, cac, gwin, sem0, sem1):
    bI = pl.program_id(0)
    j = pl.program_id(1)
    nj = pl.num_programs(1)
    r0 = j * BAND

    c_x = pltpu.make_async_copy(
        x_hbm.at[bI, pl.ds(r0, BAND), :, :], xs.at[pl.ds(1, BAND)], sem0)
    c_x.start()

    acc[...] = jnp.zeros((ACCR, W, DIM), jnp.float32)
    cac[...] = jnp.zeros((ACCR, W, 1), jnp.float32)

    lo = lo_ref[bI, j]
    hi = hi_ref[bI, j]

    def gcopy(wi):
        slot = jax.lax.rem(wi, 2)
        return pltpu.make_async_copy(gd_hbm.at[wi], gwin.at[slot],
                                     sem1.at[slot])

    @pl.when(lo < hi)
    def _():
        gcopy(lo).start()

    def body(wi, carry):
        @pl.when(wi + 1 < hi)
        def _():
            gcopy(wi + 1).start()
        gcopy(wi).wait()
        slot = jax.lax.rem(wi, 2)
        h1 = wdesc_ref[0, wi]
        w1 = wdesc_ref[1, wi]
        w1a = pl.multiple_of(jnp.minimum((w1 // 8) * 8, W - 24), 8)
        dof = w1 - w1a
        a = h1 - r0 + 16
        gd24 = jnp.concatenate(
            [gwin[slot], jnp.zeros((WIN, 8, DIM), jnp.float32)], axis=1)
        gd24 = pltpu.roll(gd24, dof, 1)
        acc[pl.ds(a, WIN), pl.ds(w1a, 24), :] += gd24
        col = jax.lax.broadcasted_iota(jnp.int32, (WIN, 24, 1), 1)
        cac[pl.ds(a, WIN), pl.ds(w1a, 24), :] += jnp.where(
            (col >= dof) & (col < dof + WIN), 1.0, 0.0)
        return carry

    jax.lax.fori_loop(lo, hi, body, 0)

    c_x.wait()

    @pl.when(j > 0)
    def _():
        t_x = pltpu.make_async_copy(
            x_hbm.at[bI, pl.ds(r0 - 1, 1), :, :], xs.at[pl.ds(0, 1)], sem0)
        t_x.start()
        t_x.wait()

    @pl.when(j == 0)
    def _():
        xs[0, :, :] = jnp.zeros((W, DIM), jnp.float32)

    @pl.when(j < nj - 1)
    def _():
        b_x = pltpu.make_async_copy(
            x_hbm.at[bI, pl.ds(r0 + BAND, 1), :, :],
            xs.at[pl.ds(BAND + 1, 1)], sem0)
        b_x.start()
        b_x.wait()

    @pl.when(j == nj - 1)
    def _():
        xs[BAND + 1, :, :] = jnp.zeros((W, DIM), jnp.float32)

    cnt3 = cac[15:15 + BAND + 2, :, :]
    integ = jnp.where(cnt3 > 0.0,
                      acc[15:15 + BAND + 2, :, :] / jnp.maximum(cnt3, 1.0),
                      0.0)
    t = xs[...] + integ
    z = jnp.zeros((BAND + 2, 1, DIM), jnp.float32)
    tl = jnp.concatenate([z, t[:, 0:W - 1, :]], axis=1)
    tr = jnp.concatenate([t[:, 1:W, :], z], axis=1)
    acc = jnp.zeros((BAND, W, DIM), jnp.float32)
    for dr in range(3):
        rows = slice(dr, dr + BAND)
        acc = (acc + tl[rows] * wd_ref[3 * dr + 0]
               + t[rows] * wd_ref[3 * dr + 1]
               + tr[rows] * wd_ref[3 * dr + 2])
    out_ref[0] = acc + bd_ref[0]


def _window_attention(x, idx, wqkv, bqkv, bias, wout, bout, interpret=False):
    grid_spec = pltpu.PrefetchScalarGridSpec(
        num_scalar_prefetch=1,
        grid=(N,),
        in_specs=[
            pl.BlockSpec(memory_space=pl.ANY),
            pl.BlockSpec((DIM, 3 * DIM), lambda i, r: (0, 0)),
            pl.BlockSpec((1, 3 * DIM), lambda i, r: (0, 0)),
            pl.BlockSpec((HEADS, P, P), lambda i, r: (0, 0, 0)),
            pl.BlockSpec((DIM, DIM), lambda i, r: (0, 0)),
            pl.BlockSpec((1, DIM), lambda i, r: (0, 0)),
        ],
        out_specs=pl.BlockSpec((1, WIN, WIN, DIM), lambda i, r: (i, 0, 0, 0)),
        scratch_shapes=[
            pltpu.VMEM((3, 17, 24, DIM), jnp.float32),
            pltpu.SemaphoreType.DMA((3,)),
        ],
    )
    return pl.pallas_call(
        _attn_body,
        grid_spec=grid_spec,
        out_shape=jax.ShapeDtypeStruct((N, WIN, WIN, DIM), jnp.float32),
        compiler_params=pltpu.CompilerParams(
            dimension_semantics=("arbitrary",)),
        interpret=interpret,
    )(idx, x, wqkv, bqkv, bias, wout, bout)


def _finalize(x, gd, wdesc, lo, hi, wd, bd, interpret=False):
    grid_spec = pltpu.PrefetchScalarGridSpec(
        num_scalar_prefetch=3,
        grid=(B, H // BAND),
        in_specs=[
            pl.BlockSpec(memory_space=pl.ANY),
            pl.BlockSpec(memory_space=pl.ANY),
            pl.BlockSpec((9, DIM), lambda b, j, *_: (0, 0)),
            pl.BlockSpec((1, DIM), lambda b, j, *_: (0, 0)),
        ],
        out_specs=pl.BlockSpec((1, BAND, W, DIM),
                               lambda b, j, *_: (b, j, 0, 0)),
        scratch_shapes=[
            pltpu.VMEM((BAND + 2, W, DIM), jnp.float32),
            pltpu.VMEM((ACCR, W, DIM), jnp.float32),
            pltpu.VMEM((ACCR, W, 1), jnp.float32),
            pltpu.VMEM((2, WIN, WIN, DIM), jnp.float32),
            pltpu.SemaphoreType.DMA,
            pltpu.SemaphoreType.DMA((2,)),
        ],
    )
    return pl.pallas_call(
        _fin_body,
        grid_spec=grid_spec,
        out_shape=jax.ShapeDtypeStruct((B, H, W, DIM), jnp.float32),
        compiler_params=pltpu.CompilerParams(
            dimension_semantics=("parallel", "arbitrary")),
        interpret=interpret,
    )(wdesc, lo, hi, x, gd, wd, bd)


def kernel(x, conv1_w, conv1_b, conv2_w, conv2_b, conv3_w, conv3_b,
           in_proj_w, in_proj_b, pe_table, out_proj_w, out_proj_b,
           out_conv_w, out_conv_b, _interpret=False):
    w = WIN
    Bn, C, Hh, Ww_ = B, DIM, H, W
    w1r = jnp.transpose(conv1_w, (2, 3, 1, 0)).reshape(9 * DIM, 4)
    sd1 = _conv1_silu(x, w1r, conv1_b.reshape(1, 4), interpret=_interpret)
    sd = jnp.transpose(sd1, (0, 3, 1, 2))
    sd = _silu(_conv2d(sd, conv2_w, conv2_b, stride=w, pad=0))
    sd = _conv2d(sd, conv3_w, conv3_b)
    Hw, Www = sd.shape[2], sd.shape[3]
    gates = jax.nn.sigmoid(sd[:, 1])
    offsets = jax.nn.sigmoid(sd[:, 2:4])
    gs = jax.nn.softmax(sd[:, 0].reshape(Bn, -1), axis=-1)
    _, topk = jax.lax.top_k(gs, KK)
    b_idx = jnp.repeat(jnp.arange(Bn), KK)
    flat = topk.reshape(-1)
    h_idx = flat // Www
    w_idx = flat % Www
    cg = gates[b_idx, h_idx, w_idx]
    hard_i = (cg > 0.5).astype(jnp.int32)
    h_off = offsets[b_idx, 0, h_idx, w_idx] * w - w / 2.0
    w_off = offsets[b_idx, 1, h_idx, w_idx] * w - w / 2.0
    h1 = jnp.clip(h_idx * w + h_off.astype(jnp.int32), 0, Hh - w)
    w1 = jnp.clip(w_idx * w + w_off.astype(jnp.int32), 0, Ww_ - w)
    sh = jnp.minimum(h1, Hh - 17)
    sw = jnp.minimum(w1, Ww_ - 17)
    key = (1 - hard_i) * (1 << 20) + b_idx * 512 + h1
    perm = jnp.argsort(key)
    key_s = key[perm]
    idx = jnp.stack([b_idx, h1, w1, sh, sw, h1 - sh, w1 - sw,
                     hard_i]).astype(jnp.int32)[:, perm]
    wdesc = idx[1:3]
    jj = jnp.arange(H // BAND) * BAND
    base = jnp.arange(Bn)[:, None] * 512
    lo = jnp.searchsorted(key_s, base + (jj[None, :] - 16)).astype(jnp.int32)
    hi = jnp.searchsorted(key_s, base + (jj[None, :] + 17)).astype(jnp.int32)

    rpi = jnp.asarray(_rel_pos_index(w).reshape(-1))
    bias = jnp.transpose(pe_table[rpi].reshape(P, P, HEADS), (2, 0, 1))
    wqkv = in_proj_w.T
    bqkv = in_proj_b.reshape(1, 3 * DIM)
    wout = out_proj_w.T
    bout = out_proj_b.reshape(1, DIM)
    wd = jnp.transpose(out_conv_w[:, 0], (1, 2, 0)).reshape(9, DIM)
    bd = out_conv_b.reshape(1, DIM)

    gd = _window_attention(x, idx, wqkv, bqkv, bias, wout, bout,
                           interpret=_interpret)
    return _finalize(x, gd, wdesc, lo, hi, wd, bd, interpret=_interpret)


# speedup vs baseline: 1.1129x; 1.1129x over previous
import jax
import jax.numpy as jnp
import numpy as np
from jax.experimental import pallas as pl
from jax.experimental.pallas import tpu as pltpu

WIN = 16
DIM = 192
HEADS = 4
B, H, W = 2, 384, 384
HD = DIM // HEADS
P = WIN * WIN
KK = int((H // WIN) * (W // WIN) * 0.25)
N = B * KK
BAND = 16
SCALE = HD ** -0.5


def _rel_pos_index(w):
    coords = np.stack(np.meshgrid(np.arange(w), np.arange(w), indexing='ij'), axis=0)
    cq = coords.reshape(2, -1)
    ck = coords.reshape(2, -1)
    rel = cq[:, :, None] - ck[:, None, :]
    rel = rel.transpose(1, 2, 0) + (w - 1)
    return (rel[:, :, 0] * (2 * w - 1) + rel[:, :, 1]).astype(np.int32)


def _conv2d(x, w, b, stride=1, pad=1, groups=1):
    out = jax.lax.conv_general_dilated(
        x, w, window_strides=(stride, stride),
        padding=[(pad, pad), (pad, pad)],
        dimension_numbers=('NCHW', 'OIHW', 'NCHW'), feature_group_count=groups)
    return out + b[None, :, None, None]


def _silu(x):
    return x * jax.nn.sigmoid(x)


def _gather_copy(idx_ref, x_hbm, rext2, sems, i, slot):
    b = idx_ref[0, i]
    sh = idx_ref[3, i]
    sw = idx_ref[4, i]
    swa = jnp.minimum((sw // 8) * 8, W - 24)
    return pltpu.make_async_copy(
        x_hbm.at[b, pl.ds(sh, 17), pl.ds(swa, 24), :],
        rext2.at[slot], sems.at[slot])


def _attn_body(idx_ref, x_hbm, wqkv_ref, bqkv_ref, bias_ref,
               wout_ref, bout_ref, gd_ref, rext2, sems):
    i = pl.program_id(0)
    sw = idx_ref[4, i]
    dh = idx_ref[5, i]
    dw = idx_ref[6, i]
    hard = idx_ref[7, i]
    swa = jnp.minimum((sw // 8) * 8, W - 24)
    doff = sw - swa
    slot = jax.lax.rem(i, 3)

    @pl.when((i == 0) & (hard == 1))
    def _():
        _gather_copy(idx_ref, x_hbm, rext2, sems, i, slot).start()

    @pl.when((i == 0) & (1 < N) & (idx_ref[7, 1] == 1))
    def _():
        _gather_copy(idx_ref, x_hbm, rext2, sems, 1, 1).start()

    @pl.when((i + 2 < N) & (idx_ref[7, i + 2] == 1))
    def _():
        nslot = jax.lax.rem(i + 2, 3)
        _gather_copy(idx_ref, x_hbm, rext2, sems, i + 2, nslot).start()

    @pl.when(hard == 1)
    def _():
        _gather_copy(idx_ref, x_hbm, rext2, sems, i, slot).wait()
        R = pltpu.roll(rext2[slot], (24 - doff) % 24, 1)[:, 0:17, :]
        Rr = jnp.concatenate([R[1:17], R[16:17]], axis=0)
        Rc = jnp.concatenate([R[:, 1:17], R[:, 16:17]], axis=1)
        Rrc = jnp.concatenate([Rc[1:17], Rc[16:17]], axis=0)
        S = R + Rc + Rr + Rrc
        p00 = S[0:16, 0:16, :]
        p01 = S[0:16, 1:17, :]
        p10 = S[1:17, 0:16, :]
        p11 = S[1:17, 1:17, :]
        patch3 = 0.25 * jnp.where(dh == 0,
                                  jnp.where(dw == 0, p00, p01),
                                  jnp.where(dw == 0, p10, p11))
        patch = patch3.reshape(P, DIM)
        qkv = jnp.dot(patch, wqkv_ref[...],
                      preferred_element_type=jnp.float32) + bqkv_ref[0, :]
        q = qkv[:, 0:DIM]
        k = qkv[:, DIM:2 * DIM]
        v = qkv[:, 2 * DIM:3 * DIM]
        outs = []
        for hh in range(HEADS):
            sl = slice(hh * HD, (hh + 1) * HD)
            sc = jax.lax.dot_general(
                q[:, sl], k[:, sl], (((1,), (1,)), ((), ())),
                preferred_element_type=jnp.float32) * SCALE + bias_ref[hh, :, :]
            m = jnp.max(sc, axis=1, keepdims=True)
            e = jnp.exp(sc - m)
            s = jnp.sum(e, axis=1, keepdims=True)
            outs.append(jnp.dot(e, v[:, sl],
                                preferred_element_type=jnp.float32) / s)
        o = jnp.concatenate(outs, axis=1)
        after = jnp.dot(o, wout_ref[...],
                        preferred_element_type=jnp.float32) + bout_ref[0, :]
        gd_ref[0] = (after - patch).reshape(WIN, WIN, DIM)


def _band_copies(x_hbm, xs2, sems, s, nj):
    bI = s // nj
    j = jax.lax.rem(s, nj)
    r0 = j * BAND
    slot = jax.lax.rem(s, 2)
    cc = pltpu.make_async_copy(
        x_hbm.at[bI, pl.ds(r0, BAND), :, :],
        xs2.at[slot, pl.ds(1, BAND)], sems.at[slot, 0])
    ct = pltpu.make_async_copy(
        x_hbm.at[bI, pl.ds(jnp.maximum(r0 - 1, 0), 1), :, :],
        xs2.at[slot, pl.ds(0, 1)], sems.at[slot, 1])
    cb = pltpu.make_async_copy(
        x_hbm.at[bI, pl.ds(jnp.minimum(r0 + BAND, H - 1), 1), :, :],
        xs2.at[slot, pl.ds(BAND + 1, 1)], sems.at[slot, 2])
    return cc, ct, cb, j


def _band_start(x_hbm, xs2, sems, s, nj):
    cc, ct, cb, j = _band_copies(x_hbm, xs2, sems, s, nj)
    cc.start()

    @pl.when(j > 0)
    def _():
        ct.start()

    @pl.when(j < nj - 1)
    def _():
        cb.start()


def _band_wait(x_hbm, xs2, sems, s, nj):
    cc, ct, cb, j = _band_copies(x_hbm, xs2, sems, s, nj)
    cc.wait()
    slot = jax.lax.rem(s, 2)

    @pl.when(j > 0)
    def _():
        ct.wait()

    @pl.when(j == 0)
    def _():
        xs2[slot, 0, :, :] = jnp.zeros((W, DIM), jnp.float32)

    @pl.when(j < nj - 1)
    def _():
        cb.wait()

    @pl.when(j == nj - 1)
    def _():
        xs2[slot, BAND + 1, :, :] = jnp.zeros((W, DIM), jnp.float32)


def _score_body(x_hbm, w1_ref, b1_ref, out_ref, xs2, sems):
    bI = pl.program_id(0)
    j = pl.program_id(1)
    nj = pl.num_programs(1)
    s = bI * nj + j
    slot = jax.lax.rem(s, 2)

    @pl.when(s == 0)
    def _():
        _band_start(x_hbm, xs2, sems, s, nj)

    @pl.when(s + 1 < B * nj)
    def _():
        _band_start(x_hbm, xs2, sems, s + 1, nj)

    _band_wait(x_hbm, xs2, sems, s, nj)

    t = xs2[slot]
    z = jnp.zeros((BAND + 2, 1, DIM), jnp.float32)
    tl = jnp.concatenate([z, t[:, 0:W - 1, :]], axis=1)
    tr = jnp.concatenate([t[:, 1:W, :], z], axis=1)
    acc = jnp.zeros((BAND * W, 4), jnp.float32)
    for dr in range(3):
        rows = slice(dr, dr + BAND)
        for dc, src in ((0, tl), (1, t), (2, tr)):
            acc = acc + jnp.dot(src[rows].reshape(BAND * W, DIM),
                                w1_ref[(3 * dr + dc) * DIM:
                                       (3 * dr + dc + 1) * DIM, :],
                                preferred_element_type=jnp.float32)
    acc = acc + b1_ref[0]
    out_ref[0] = (acc * jax.nn.sigmoid(acc)).reshape(BAND, W, 4)


def _conv1_silu(x, w1r, b1, interpret=False):
    grid_spec = pltpu.PrefetchScalarGridSpec(
        num_scalar_prefetch=0,
        grid=(B, H // BAND),
        in_specs=[
            pl.BlockSpec(memory_space=pl.ANY),
            pl.BlockSpec((9 * DIM, 4), lambda b, j: (0, 0)),
            pl.BlockSpec((1, 4), lambda b, j: (0, 0)),
        ],
        out_specs=pl.BlockSpec((1, BAND, W, 4), lambda b, j: (b, j, 0, 0)),
        scratch_shapes=[
            pltpu.VMEM((2, BAND + 2, W, DIM), jnp.float32),
            pltpu.SemaphoreType.DMA((2, 3)),
        ],
    )
    return pl.pallas_call(
        _score_body,
        grid_spec=grid_spec,
        out_shape=jax.ShapeDtypeStruct((B, H, W, 4), jnp.float32),
        compiler_params=pltpu.CompilerParams(
            dimension_semantics=("arbitrary", "arbitrary")),
        interpret=interpret,
    )(x, w1r, b1)


ACCR = BAND + 32


def _fin_body(wdesc_ref, lo_ref, hi_ref, x_hbm, gd_hbm, wd_ref, bd_ref,
              out_ref, xs, acc, cac, gwin, sem0, sem1):
    bI = pl.program_id(0)
    j = pl.program_id(1)
    nj = pl.num_programs(1)
    r0 = j * BAND

    c_x = pltpu.make_async_copy(
        x_hbm.at[bI, pl.ds(r0, BAND), :, :], xs.at[pl.ds(1, BAND)], sem0)
    c_x.start()

    acc[...] = jnp.zeros((ACCR, W, DIM), jnp.float32)
    cac[...] = jnp.zeros((ACCR, W, 1), jnp.float32)

    lo = lo_ref[bI, j]
    hi = hi_ref[bI, j]

    def gcopy(wi):
        slot = jax.lax.rem(wi, 2)
        return pltpu.make_async_copy(gd_hbm.at[wi], gwin.at[slot],
                                     sem1.at[slot])

    @pl.when(lo < hi)
    def _():
        gcopy(lo).start()

    def body(wi, carry):
        @pl.when(wi + 1 < hi)
        def _():
            gcopy(wi + 1).start()
        gcopy(wi).wait()
        slot = jax.lax.rem(wi, 2)
        h1 = wdesc_ref[0, wi]
        w1 = wdesc_ref[1, wi]
        w1a = pl.multiple_of(jnp.minimum((w1 // 8) * 8, W - 24), 8)
        dof = w1 - w1a
        a = h1 - r0 + 16
        gd24 = jnp.concatenate(
            [gwin[slot], jnp.zeros((WIN, 8, DIM), jnp.float32)], axis=1)
        gd24 = pltpu.roll(gd24, dof, 1)
        acc[pl.ds(a, WIN), pl.ds(w1a, 24), :] += gd24
        col = jax.lax.broadcasted_iota(jnp.int32, (WIN, 24, 1), 1)
        cac[pl.ds(a, WIN), pl.ds(w1a, 24), :] += jnp.where(
            (col >= dof) & (col < dof + WIN), 1.0, 0.0)
        return carry

    jax.lax.fori_loop(lo, hi, body, 0)

    c_x.wait()

    @pl.when(j > 0)
    def _():
        t_x = pltpu.make_async_copy(
            x_hbm.at[bI, pl.ds(r0 - 1, 1), :, :], xs.at[pl.ds(0, 1)], sem0)
        t_x.start()
        t_x.wait()

    @pl.when(j == 0)
    def _():
        xs[0, :, :] = jnp.zeros((W, DIM), jnp.float32)

    @pl.when(j < nj - 1)
    def _():
        b_x = pltpu.make_async_copy(
            x_hbm.at[bI, pl.ds(r0 + BAND, 1), :, :],
            xs.at[pl.ds(BAND + 1, 1)], sem0)
        b_x.start()
        b_x.wait()

    @pl.when(j == nj - 1)
    def _():
        xs[BAND + 1, :, :] = jnp.zeros((W, DIM), jnp.float32)

    cnt3 = cac[15:15 + BAND + 2, :, :]
    integ = jnp.where(cnt3 > 0.0,
                      acc[15:15 + BAND + 2, :, :] / jnp.maximum(cnt3, 1.0),
                      0.0)
    t = xs[...] + integ
    z = jnp.zeros((BAND + 2, 1, DIM), jnp.float32)
    tl = jnp.concatenate([z, t[:, 0:W - 1, :]], axis=1)
    tr = jnp.concatenate([t[:, 1:W, :], z], axis=1)
    acc = jnp.zeros((BAND, W, DIM), jnp.float32)
    for dr in range(3):
        rows = slice(dr, dr + BAND)
        acc = (acc + tl[rows] * wd_ref[3 * dr + 0]
               + t[rows] * wd_ref[3 * dr + 1]
               + tr[rows] * wd_ref[3 * dr + 2])
    out_ref[0] = acc + bd_ref[0]


def _window_attention(x, idx, wqkv, bqkv, bias, wout, bout, interpret=False):
    grid_spec = pltpu.PrefetchScalarGridSpec(
        num_scalar_prefetch=1,
        grid=(N,),
        in_specs=[
            pl.BlockSpec(memory_space=pl.ANY),
            pl.BlockSpec((DIM, 3 * DIM), lambda i, r: (0, 0)),
            pl.BlockSpec((1, 3 * DIM), lambda i, r: (0, 0)),
            pl.BlockSpec((HEADS, P, P), lambda i, r: (0, 0, 0)),
            pl.BlockSpec((DIM, DIM), lambda i, r: (0, 0)),
            pl.BlockSpec((1, DIM), lambda i, r: (0, 0)),
        ],
        out_specs=pl.BlockSpec((1, WIN, WIN, DIM), lambda i, r: (i, 0, 0, 0)),
        scratch_shapes=[
            pltpu.VMEM((3, 17, 24, DIM), jnp.float32),
            pltpu.SemaphoreType.DMA((3,)),
        ],
    )
    return pl.pallas_call(
        _attn_body,
        grid_spec=grid_spec,
        out_shape=jax.ShapeDtypeStruct((N, WIN, WIN, DIM), jnp.float32),
        compiler_params=pltpu.CompilerParams(
            dimension_semantics=("arbitrary",)),
        interpret=interpret,
    )(idx, x, wqkv, bqkv, bias, wout, bout)


def _finalize(x, gd, wdesc, lo, hi, wd, bd, interpret=False):
    grid_spec = pltpu.PrefetchScalarGridSpec(
        num_scalar_prefetch=3,
        grid=(B, H // BAND),
        in_specs=[
            pl.BlockSpec(memory_space=pl.ANY),
            pl.BlockSpec(memory_space=pl.ANY),
            pl.BlockSpec((9, DIM), lambda b, j, *_: (0, 0)),
            pl.BlockSpec((1, DIM), lambda b, j, *_: (0, 0)),
        ],
        out_specs=pl.BlockSpec((1, BAND, W, DIM),
                               lambda b, j, *_: (b, j, 0, 0)),
        scratch_shapes=[
            pltpu.VMEM((BAND + 2, W, DIM), jnp.float32),
            pltpu.VMEM((ACCR, W, DIM), jnp.float32),
            pltpu.VMEM((ACCR, W, 1), jnp.float32),
            pltpu.VMEM((2, WIN, WIN, DIM), jnp.float32),
            pltpu.SemaphoreType.DMA,
            pltpu.SemaphoreType.DMA((2,)),
        ],
    )
    return pl.pallas_call(
        _fin_body,
        grid_spec=grid_spec,
        out_shape=jax.ShapeDtypeStruct((B, H, W, DIM), jnp.float32),
        compiler_params=pltpu.CompilerParams(
            dimension_semantics=("parallel", "arbitrary")),
        interpret=interpret,
    )(wdesc, lo, hi, x, gd, wd, bd)


def kernel(x, conv1_w, conv1_b, conv2_w, conv2_b, conv3_w, conv3_b,
           in_proj_w, in_proj_b, pe_table, out_proj_w, out_proj_b,
           out_conv_w, out_conv_b, _interpret=False):
    w = WIN
    Bn, C, Hh, Ww_ = B, DIM, H, W
    w1r = jnp.transpose(conv1_w, (2, 3, 1, 0)).reshape(9 * DIM, 4)
    sd1 = _conv1_silu(x, w1r, conv1_b.reshape(1, 4), interpret=_interpret)
    sd = jnp.transpose(sd1, (0, 3, 1, 2))
    sd = _silu(_conv2d(sd, conv2_w, conv2_b, stride=w, pad=0))
    sd = _conv2d(sd, conv3_w, conv3_b)
    Hw, Www = sd.shape[2], sd.shape[3]
    gates = jax.nn.sigmoid(sd[:, 1])
    offsets = jax.nn.sigmoid(sd[:, 2:4])
    gs = jax.nn.softmax(sd[:, 0].reshape(Bn, -1), axis=-1)
    _, topk = jax.lax.top_k(gs, KK)
    b_idx = jnp.repeat(jnp.arange(Bn), KK)
    flat = topk.reshape(-1)
    h_idx = flat // Www
    w_idx = flat % Www
    cg = gates[b_idx, h_idx, w_idx]
    hard_i = (cg > 0.5).astype(jnp.int32)
    h_off = offsets[b_idx, 0, h_idx, w_idx] * w - w / 2.0
    w_off = offsets[b_idx, 1, h_idx, w_idx] * w - w / 2.0
    h1 = jnp.clip(h_idx * w + h_off.astype(jnp.int32), 0, Hh - w)
    w1 = jnp.clip(w_idx * w + w_off.astype(jnp.int32), 0, Ww_ - w)
    sh = jnp.minimum(h1, Hh - 17)
    sw = jnp.minimum(w1, Ww_ - 17)
    key = (1 - hard_i) * (1 << 20) + b_idx * 512 + h1
    perm = jnp.argsort(key)
    key_s = key[perm]
    idx = jnp.stack([b_idx, h1, w1, sh, sw, h1 - sh, w1 - sw,
                     hard_i]).astype(jnp.int32)[:, perm]
    wdesc = idx[1:3]
    jj = jnp.arange(H // BAND) * BAND
    base = jnp.arange(Bn)[:, None] * 512
    lo = jnp.searchsorted(key_s, base + (jj[None, :] - 16)).astype(jnp.int32)
    hi = jnp.searchsorted(key_s, base + (jj[None, :] + 17)).astype(jnp.int32)

    rpi = jnp.asarray(_rel_pos_index(w).reshape(-1))
    bias = jnp.transpose(pe_table[rpi].reshape(P, P, HEADS), (2, 0, 1))
    wqkv = in_proj_w.T
    bqkv = in_proj_b.reshape(1, 3 * DIM)
    wout = out_proj_w.T
    bout = out_proj_b.reshape(1, DIM)
    wd = jnp.transpose(out_conv_w[:, 0], (1, 2, 0)).reshape(9, DIM)
    bd = out_conv_b.reshape(1, DIM)

    gd = _window_attention(x, idx, wqkv, bqkv, bias, wout, bout,
                           interpret=_interpret)
    return _finalize(x, gd, wdesc, lo, hi, wd, bd, interpret=_interpret)
